# baseline (speedup 1.0000x reference)
import functools

import jax
import jax.numpy as jnp
from jax import lax
from jax.experimental import pallas as pl
from jax.experimental.pallas import tpu as pltpu
from jax.experimental.pallas import tpu_sc as plsc

LATENT = 64


@functools.cache
def _make_sc_gather(B: int, D: int):
    info = plsc.get_sparse_core_info()
    NC, NS = info.num_cores, info.num_subcores
    NW = NC * NS
    assert B % (8 * NW) == 0
    b_per_w = B // NW
    n_grp = b_per_w // 16

    mesh = plsc.VectorSubcoreMesh(core_axis_name="c", subcore_axis_name="s")

    @functools.partial(
        pl.kernel,
        mesh=mesh,
        compiler_params=pltpu.CompilerParams(needs_layout_passes=False),
        out_type=[
            jax.ShapeDtypeStruct((B * D,), jnp.int32),
            jax.ShapeDtypeStruct((B * D,), jnp.int32),
        ],
        scratch_types=[
            pltpu.VMEM((b_per_w,), jnp.int32),
            pltpu.VMEM((b_per_w * D,), jnp.int32),
            pltpu.SemaphoreType.DMA,
        ],
    )
    def gather(user_hbm, item_hbm, uemb_hbm, iemb_hbm, u_out, i_out,
               idx_v, rows_v, sem):
        wid = lax.axis_index("s") * NC + lax.axis_index("c")
        base = wid * b_per_w

        L = 16

        def one_table(which_idx_hbm, emb_hbm, out_hbm):
            pltpu.sync_copy(which_idx_hbm.at[pl.ds(base, b_per_w)], idx_v)

            def fire(g):
                v = idx_v[pl.ds(g * L, L)] * D
                for l in range(L):
                    r = pl.multiple_of(v[l], D)
                    pltpu.async_copy(
                        emb_hbm.at[pl.ds(r, D)],
                        rows_v.at[pl.ds(pl.multiple_of((g * L + l) * D, D), D)],
                        sem)

            def drain():
                for _ in range(L):
                    pltpu.make_async_copy(
                        emb_hbm.at[pl.ds(0, D)],
                        rows_v.at[pl.ds(0, D)], sem).wait()

            fire(0)

            def body(g, _):
                @pl.when(g + 1 < n_grp)
                def _():
                    fire(g + 1)

                drain()
                return 0

            lax.fori_loop(0, n_grp, body, 0)
            pltpu.sync_copy(
                rows_v,
                out_hbm.at[pl.ds(pl.multiple_of(base * D, D), b_per_w * D)])

        one_table(user_hbm, uemb_hbm, u_out)
        one_table(item_hbm, iemb_hbm, i_out)

    return gather


def _mlp_body(u_ref, i_ref, w1u_ref, w1i_ref, b1_ref, w2_ref, b2_ref,
              w3_ref, b3_ref, out_ref):
    u = u_ref[...].astype(jnp.float32)
    i = i_ref[...].astype(jnp.float32)
    x = jnp.dot(u, w1u_ref[...], preferred_element_type=jnp.float32)
    x = x + jnp.dot(i, w1i_ref[...], preferred_element_type=jnp.float32)
    h = jnp.maximum(x + b1_ref[...], 0.0)
    h = jnp.maximum(
        jnp.dot(h, w2_ref[...], preferred_element_type=jnp.float32) + b2_ref[...],
        0.0,
    )
    out_ref[...] = (
        jnp.dot(h, w3_ref[...], preferred_element_type=jnp.float32) + b3_ref[...]
    )


def _mlp(u, i, w1uT, w1iT, b1, w2T, b2, w3T, b3, blk: int):
    B, D = u.shape
    H1 = w1uT.shape[1]
    H2 = w2T.shape[1]
    grid = (B // blk,)
    return pl.pallas_call(
        _mlp_body,
        grid=grid,
        in_specs=[
            pl.BlockSpec((blk, D), lambda g: (g, 0)),
            pl.BlockSpec((blk, D), lambda g: (g, 0)),
            pl.BlockSpec((D, H1), lambda g: (0, 0)),
            pl.BlockSpec((D, H1), lambda g: (0, 0)),
            pl.BlockSpec((1, H1), lambda g: (0, 0)),
            pl.BlockSpec((H1, H2), lambda g: (0, 0)),
            pl.BlockSpec((1, H2), lambda g: (0, 0)),
            pl.BlockSpec((H2, 1), lambda g: (0, 0)),
            pl.BlockSpec((1, 1), lambda g: (0, 0)),
        ],
        out_specs=pl.BlockSpec((blk, 1), lambda g: (g, 0)),
        out_shape=jax.ShapeDtypeStruct((B, 1), jnp.float32),
    )(u, i, w1uT, w1iT, b1, w2T, b2, w3T, b3)


def _to_flat_i32(emb):
    V, D = emb.shape
    b = emb.astype(jnp.bfloat16).reshape(V * D // 2, 2)
    return lax.bitcast_convert_type(b, jnp.int32)


def _from_flat_i32(flat, B, D):
    b = lax.bitcast_convert_type(flat.reshape(B, D // 2), jnp.bfloat16)
    return b.reshape(B, D)


def kernel(user, item, user_emb, item_emb, W1, b1, W2, b2, W3, b3):
    B = user.shape[0]
    D = user_emb.shape[1]
    uf, itf = _make_sc_gather(B, D // 2)(
        user, item, _to_flat_i32(user_emb), _to_flat_i32(item_emb))
    u = _from_flat_i32(uf, B, D)
    i = _from_flat_i32(itf, B, D)
    w1T = W1.T
    out = _mlp(
        u, i,
        w1T[:D], w1T[D:],
        b1.reshape(1, -1),
        W2.T,
        b2.reshape(1, -1),
        W3.T,
        b3.reshape(1, 1),
        blk=2048,
    )
    return out[:, 0]


# speedup vs baseline: 61.0568x; 61.0568x over previous
import functools

import jax
import jax.numpy as jnp
from jax import lax
from jax.experimental import pallas as pl
from jax.experimental.pallas import tpu as pltpu
from jax.experimental.pallas import tpu_sc as plsc

LATENT = 64


@functools.cache
def _make_sc_gather(B: int, D: int, tag: str):
    info = plsc.get_sparse_core_info()
    NC, NS = info.num_cores, info.num_subcores
    NW = NC * NS
    assert B % (8 * NW) == 0
    b_per_w = B // NW
    n_grp = b_per_w // 16

    mesh = plsc.VectorSubcoreMesh(core_axis_name="c", subcore_axis_name="s")

    @functools.partial(
        pl.kernel,
        mesh=mesh,
        name=f"gather_{tag}",
        out_type=jax.ShapeDtypeStruct((B, D), jnp.float32),
        scratch_types=[
            pltpu.VMEM((b_per_w,), jnp.int32),
            pltpu.VMEM((b_per_w, D), jnp.float32),
            pltpu.SemaphoreType.DMA,
        ],
    )
    def gather(idx_hbm, emb_hbm, out_hbm, idx_v, rows_v, sem):
        wid = lax.axis_index("s") * NC + lax.axis_index("c")
        base = wid * b_per_w

        L = 16
        pltpu.sync_copy(idx_hbm.at[pl.ds(base, b_per_w)], idx_v)

        def fire(g):
            v = idx_v[pl.ds(g * L, L)]
            for l in range(L):
                r = v[l]
                pltpu.async_copy(
                    emb_hbm.at[pl.ds(r, 1)],
                    rows_v.at[pl.ds(g * L + l, 1)], sem)

        def drain():
            for _ in range(L):
                pltpu.make_async_copy(
                    emb_hbm.at[pl.ds(0, 1)],
                    rows_v.at[pl.ds(0, 1)], sem).wait()

        fire(0)

        def body(g, _):
            @pl.when(g + 1 < n_grp)
            def _():
                fire(g + 1)

            drain()
            return 0

        lax.fori_loop(0, n_grp, body, 0)
        pltpu.sync_copy(rows_v, out_hbm.at[pl.ds(base, b_per_w)])

    return gather


def _mlp_body(u_ref, i_ref, w1u_ref, w1i_ref, b1_ref, w2_ref, b2_ref,
              w3_ref, b3_ref, out_ref):
    x = jnp.dot(u_ref[...], w1u_ref[...], preferred_element_type=jnp.float32)
    x = x + jnp.dot(i_ref[...], w1i_ref[...], preferred_element_type=jnp.float32)
    h = jnp.maximum(x + b1_ref[...], 0.0)
    h = jnp.maximum(
        jnp.dot(h, w2_ref[...], preferred_element_type=jnp.float32) + b2_ref[...],
        0.0,
    )
    out_ref[...] = (
        jnp.dot(h, w3_ref[...], preferred_element_type=jnp.float32) + b3_ref[...]
    )


def _mlp(u, i, w1uT, w1iT, b1, w2T, b2, w3T, b3, blk: int):
    B, D = u.shape
    H1 = w1uT.shape[1]
    H2 = w2T.shape[1]
    grid = (B // blk,)
    return pl.pallas_call(
        _mlp_body,
        grid=grid,
        in_specs=[
            pl.BlockSpec((blk, D), lambda g: (g, 0)),
            pl.BlockSpec((blk, D), lambda g: (g, 0)),
            pl.BlockSpec((D, H1), lambda g: (0, 0)),
            pl.BlockSpec((D, H1), lambda g: (0, 0)),
            pl.BlockSpec((1, H1), lambda g: (0, 0)),
            pl.BlockSpec((H1, H2), lambda g: (0, 0)),
            pl.BlockSpec((1, H2), lambda g: (0, 0)),
            pl.BlockSpec((H2, 1), lambda g: (0, 0)),
            pl.BlockSpec((1, 1), lambda g: (0, 0)),
        ],
        out_specs=pl.BlockSpec((blk, 1), lambda g: (g, 0)),
        out_shape=jax.ShapeDtypeStruct((B, 1), jnp.float32),
    )(u, i, w1uT, w1iT, b1, w2T, b2, w3T, b3)


def kernel(user, item, user_emb, item_emb, W1, b1, W2, b2, W3, b3):
    B = user.shape[0]
    D = user_emb.shape[1]
    u = _make_sc_gather(B, D, "user")(user, user_emb)
    i = _make_sc_gather(B, D, "item")(item, item_emb)
    w1T = W1.T
    out = _mlp(
        u, i,
        w1T[:D], w1T[D:],
        b1.reshape(1, -1),
        W2.T,
        b2.reshape(1, -1),
        W3.T,
        b3.reshape(1, 1),
        blk=2048,
    )
    return out[:, 0]
